# baseline (speedup 1.0000x reference)
import jax
import jax.numpy as jnp
from jax.experimental import pallas as pl
from jax.experimental.pallas import tpu as pltpu

_CP = 256
_TM = 512


def _fused_square_kernel(w_hbm, o_hbm, wbf, in_buf, out_buf, in_sem, out_sem):
    F = w_hbm.shape[0]
    n_cast = F // _CP
    n_out = F // _TM

    def in_dma(slot, p):
        return pltpu.make_async_copy(
            w_hbm.at[pl.ds(p * _CP, _CP)], in_buf.at[slot], in_sem.at[slot])

    def out_dma(slot, i):
        return pltpu.make_async_copy(
            out_buf.at[slot], o_hbm.at[pl.ds(i * _TM, _TM)], out_sem.at[slot])

    in_dma(0, 0).start()

    def cast_body(p, _):
        cur = jax.lax.rem(p, 2)
        nxt = jax.lax.rem(p + 1, 2)

        @pl.when(p + 1 < n_cast)
        def _():
            in_dma(nxt, p + 1).start()

        in_dma(cur, 0).wait()
        wbf[pl.ds(p * _CP, _CP), :] = in_buf[cur].astype(jnp.bfloat16)
        return ()

    jax.lax.fori_loop(0, n_cast, cast_body, ())

    _TN = 512
    nb_per_tile = F // _TN

    def mm_body(s, _):
        i = s // nb_per_tile
        nb = jax.lax.rem(s, nb_per_tile)
        cur = jax.lax.rem(i, 2)

        @pl.when(jnp.logical_and(nb == 0, i >= 2))
        def _():
            out_dma(cur, 0).wait()

        a = wbf[pl.ds(i * _TM, _TM), :]
        ob = out_buf.at[cur]
        ob[:, pl.ds(nb * _TN, _TN)] = jnp.dot(
            a, wbf[:, pl.ds(nb * _TN, _TN)], preferred_element_type=jnp.float32)

        @pl.when(nb == nb_per_tile - 1)
        def _():
            out_dma(cur, i).start()

        return ()

    jax.lax.fori_loop(0, n_out * nb_per_tile, mm_body, ())
    out_dma((n_out - 2) % 2, 0).wait()
    out_dma((n_out - 1) % 2, 0).wait()


def kernel(weight):
    B, F, F2 = weight.shape
    assert B == 1 and F == F2 and F % 512 == 0 and F * F * 2 <= (32 << 20)
    w2d = weight[0]
    out2d = pl.pallas_call(
        _fused_square_kernel,
        out_shape=jax.ShapeDtypeStruct((F, F), jnp.float32),
        in_specs=[pl.BlockSpec(memory_space=pl.ANY)],
        out_specs=pl.BlockSpec(memory_space=pl.ANY),
        scratch_shapes=[
            pltpu.VMEM((F, F), jnp.bfloat16),
            pltpu.VMEM((2, _CP, F), jnp.float32),
            pltpu.VMEM((2, _TM, F), jnp.float32),
            pltpu.SemaphoreType.DMA((2,)),
            pltpu.SemaphoreType.DMA((2,)),
        ],
        compiler_params=pltpu.CompilerParams(
            vmem_limit_bytes=62 << 20,
        ),
        cost_estimate=pl.CostEstimate(
            flops=2 * F**3,
            transcendentals=0,
            bytes_accessed=2 * F * F * 4,
        ),
    )(w2d)
    return out2d[None, :, :]


# speedup vs baseline: 1.0898x; 1.0898x over previous
import jax
import jax.numpy as jnp
from jax.experimental import pallas as pl
from jax.experimental.pallas import tpu as pltpu

_TM = 512


def _fused_square_kernel(w_hbm, o_hbm, wbf, out_buf, in_sem, out_sem):
    F = w_hbm.shape[0]
    n_tiles = F // _TM

    def in_dma(slot, p):
        return pltpu.make_async_copy(
            w_hbm.at[pl.ds(p * _TM, _TM)], out_buf.at[slot], in_sem.at[slot])

    def out_dma(slot, i):
        return pltpu.make_async_copy(
            out_buf.at[slot], o_hbm.at[pl.ds(i * _TM, _TM)], out_sem.at[slot])

    in_dma(0, 0).start()

    def cast_body(p, _):
        cur = jax.lax.rem(p, 2)
        nxt = jax.lax.rem(p + 1, 2)

        @pl.when(p + 1 < n_tiles)
        def _():
            in_dma(nxt, p + 1).start()

        in_dma(cur, 0).wait()
        wbf[pl.ds(p * _TM, _TM), :] = out_buf[cur].astype(jnp.bfloat16)
        return ()

    jax.lax.fori_loop(0, n_tiles, cast_body, ())

    def mm_body(i, _):
        cur = jax.lax.rem(i, 2)

        @pl.when(i >= 2)
        def _():
            out_dma(cur, 0).wait()

        a = wbf[pl.ds(i * _TM, _TM), :]
        ob = out_buf.at[cur]
        ob[...] = jnp.dot(a, wbf[...], preferred_element_type=jnp.float32)
        out_dma(cur, i).start()
        return ()

    jax.lax.fori_loop(0, n_tiles, mm_body, ())
    out_dma((n_tiles - 2) % 2, 0).wait()
    out_dma((n_tiles - 1) % 2, 0).wait()


def kernel(weight):
    B, F, F2 = weight.shape
    assert B == 1 and F == F2 and F % (2 * _TM) == 0 and F * F * 2 <= (32 << 20)
    w2d = weight[0]
    out2d = pl.pallas_call(
        _fused_square_kernel,
        out_shape=jax.ShapeDtypeStruct((F, F), jnp.float32),
        in_specs=[pl.BlockSpec(memory_space=pl.ANY)],
        out_specs=pl.BlockSpec(memory_space=pl.ANY),
        scratch_shapes=[
            pltpu.VMEM((F, F), jnp.bfloat16),
            pltpu.VMEM((2, _TM, F), jnp.float32),
            pltpu.SemaphoreType.DMA((2,)),
            pltpu.SemaphoreType.DMA((2,)),
        ],
        compiler_params=pltpu.CompilerParams(
            vmem_limit_bytes=62 << 20,
        ),
        cost_estimate=pl.CostEstimate(
            flops=2 * F**3,
            transcendentals=0,
            bytes_accessed=2 * F * F * 4,
        ),
    )(w2d)
    return out2d[None, :, :]
